# baseline (speedup 1.0000x reference)
import functools

import jax
import jax.numpy as jnp
import numpy as np
from jax import lax
from jax.experimental import pallas as pl
from jax.experimental.pallas import tpu as pltpu
from jax.experimental.pallas import tpu_sc as plsc

D = 64
NC, NS, L = 2, 16, 16
NW = NC * NS
CHUNK = 128
NFI = 8 * 16

_PERM = np.empty((D,), dtype=np.int32)
for _h in (0, 1):
    for _i in range(16):
        _PERM[_h * 32 + _i] = _h * 32 + 2 * _i
        _PERM[_h * 32 + 16 + _i] = _h * 32 + 2 * _i + 1


def _widen(v32):
    vi = plsc.bitcast(v32, jnp.int32)
    ev = plsc.bitcast(lax.shift_left(vi, 16), jnp.float32)
    od = plsc.bitcast(
        lax.bitwise_and(vi, jnp.int32(-65536)), jnp.float32)
    return ev, od


@functools.cache
def _sc_embed(n_tokens):
    n_per_w = n_tokens // NW
    n_chunks = n_per_w // CHUNK
    mesh = plsc.VectorSubcoreMesh(core_axis_name="c", subcore_axis_name="s")

    @functools.partial(
        pl.kernel,
        out_type=jax.ShapeDtypeStruct((n_tokens, D // 2), jnp.int32),
        mesh=mesh,
        compiler_params=pltpu.CompilerParams(use_tc_tiling_on_sc=False,
                                             needs_layout_passes=False),
        scratch_types=[
            pltpu.VMEM((2, 5, CHUNK), jnp.int32),
            pltpu.VMEM((2, 2, CHUNK, D), jnp.bfloat16),
            pltpu.VMEM((2, CHUNK, D), jnp.float32),
            pltpu.VMEM((2, CHUNK, D // 2), jnp.int32),
            pltpu.VMEM((8, D), jnp.float32),
            pltpu.VMEM((16, D), jnp.float32),
            pltpu.VMEM((NFI, D), jnp.float32),
            pltpu.VMEM_SHARED((4102, D), jnp.bfloat16),
            pltpu.VMEM_SHARED((4102, D), jnp.bfloat16),
            pltpu.SemaphoreType.DMA,
            pltpu.SemaphoreType.DMA,
            pltpu.SemaphoreType.DMA,
            pltpu.SemaphoreType.DMA,
        ],
    )
    def k(comb_hbm, wcx, wcy, wsf, wsi, out_hbm, ib, rows, acc, outb,
          wsf_v, wsi_v, wfi, wcx_sh, wcy_sh, gsem0, gsem1, osem0, osem1):
        wid = lax.axis_index("s") * NC + lax.axis_index("c")
        w_chunk0 = wid * n_chunks
        w_base = wid * n_per_w
        gsems = [gsem0, gsem1]
        osems = [osem0, osem1]

        @pl.when(lax.axis_index("s") == 0)
        def _stage():
            pltpu.sync_copy(wcx, wcx_sh)
            pltpu.sync_copy(wcy, wcy_sh)

        pltpu.sync_copy(wsf, wsf_v)
        pltpu.sync_copy(wsi, wsi_v)

        def fuse_body(r, c):
            f = lax.shift_right_logical(r, 4)
            i = lax.bitwise_and(r, 15)
            for kk in range(D // L):
                sl = pl.ds(kk * L, L)
                wfi[r, sl] = wsf_v[f, sl] + wsi_v[i, sl]
            return c

        lax.fori_loop(0, NFI, fuse_body, 0)
        plsc.subcore_barrier()

        def load_idx_sync(j, b):
            pltpu.sync_copy(comb_hbm.at[w_chunk0 + j], ib.at[b])

        def mask_idx(b):
            for kk in range(CHUNK // L):
                sl = pl.ds(kk * L, L)
                a = ib[b, 4, sl]
                ib[b, 0, sl] = ib[b, 0, sl] * a
                ib[b, 1, sl] = ib[b, 1, sl] * a
                ib[b, 2, sl] = ib[b, 2, sl] * 16 + ib[b, 3, sl]

        def fire_gathers(b):
            sem = gsems[b]
            pltpu.async_copy(wcx_sh.at[ib.at[b, 0]], rows.at[b, 0], sem)
            pltpu.async_copy(wcy_sh.at[ib.at[b, 1]], rows.at[b, 1], sem)

        def drain_gathers(b):
            sem = gsems[b]
            for t in range(2):
                pltpu.make_async_copy(wcx.at[ib.at[b, t]], rows.at[b, t],
                                      sem).wait()

        def fi_scatter(b):
            lane = lax.iota(jnp.int32, L)

            def fi_body(g, c):
                cvec = ib[b, 2, pl.ds(g * L, L)]
                tok = g * L + lane
                for d in range(D):
                    dvec = lax.full((L,), d, jnp.int32)
                    val = plsc.load_gather(wfi, [cvec, dvec])
                    plsc.store_scatter(acc.at[b], [tok, dvec], val)
                return c

            lax.fori_loop(0, CHUNK // L, fi_body, 0)

        def sum_pack(b):
            rnd = jnp.int32(32768)
            himask = jnp.int32(-65536)

            def body(q, c):
                for rr in range(4):
                    r = q * 4 + rr
                    for h in range(2):
                        vx = rows[b, 0, r, pl.ds(h * 32, 32)]
                        vy = rows[b, 1, r, pl.ds(h * 32, 32)]
                        xe, xo = _widen(vx)
                        ye, yo = _widen(vy)
                        se = acc[b, r, pl.ds(h * 32, L)] + xe + ye
                        so = acc[b, r, pl.ds(h * 32 + L, L)] + xo + yo
                        ei = plsc.bitcast(se, jnp.int32) + rnd
                        oi = plsc.bitcast(so, jnp.int32) + rnd
                        er = lax.shift_right_logical(ei, 16)
                        orr = lax.bitwise_and(oi, himask)
                        outb[b, r, pl.ds(h * L, L)] = lax.bitwise_or(er, orr)
                return c

            lax.fori_loop(0, CHUNK // 4, body, 0)

        def fire_out(j, b):
            base = w_base + j * CHUNK
            pltpu.async_copy(outb.at[b], out_hbm.at[pl.ds(base, CHUNK)],
                             osems[b])

        def drain_out(b):
            pltpu.make_async_copy(outb.at[b],
                                  out_hbm.at[pl.ds(0, CHUNK)],
                                  osems[b]).wait()

        for b in range(2):
            load_idx_sync(b, b)
            mask_idx(b)
            fire_gathers(b)
            fi_scatter(b)

        def pair_body(t, carry):
            for b in range(2):
                j = t * 2 + b
                drain_gathers(b)
                sum_pack(b)
                fire_out(j, b)
                load_idx_sync(j + 2, b)
                mask_idx(b)
                fire_gathers(b)
                fi_scatter(b)
                drain_out(b)
            return carry

        lax.fori_loop(0, (n_chunks - 2) // 2, pair_body, 0)

        for b in range(2):
            j = n_chunks - 2 + b
            drain_gathers(b)
            sum_pack(b)
            fire_out(j, b)
        for b in range(2):
            drain_out(b)

    return k


def kernel(cad_vec, flag_vec, index_vec, key_padding_mask, W_cx, W_cy, W_sf,
           W_si):
    B, S = flag_vec.shape
    n = B * S
    nck = n // CHUNK
    x = cad_vec[:, :, 0].reshape(nck, CHUNK)
    y = cad_vec[:, :, 1].reshape(nck, CHUNK)
    fl = flag_vec.reshape(nck, CHUNK)
    iv = index_vec.reshape(nck, CHUNK)
    act = (~key_padding_mask).reshape(nck, CHUNK).astype(jnp.int32)
    comb = jnp.stack([x, y, fl, iv, act], axis=1)
    perm = jnp.asarray(_PERM)
    wcx = W_cx.astype(jnp.bfloat16)
    wcy = W_cy.astype(jnp.bfloat16)
    wsf = W_sf[:, perm]
    wsi = W_si[:, perm]
    out_i32 = _sc_embed(n)(comb, wcx, wcy, wsf, wsi)
    out_bf16 = lax.bitcast_convert_type(out_i32, jnp.bfloat16)
    out = out_bf16.astype(jnp.float32).reshape(n, D)
    return out.reshape(B, S, D)


# speedup vs baseline: 1.9532x; 1.9532x over previous
import functools

import jax
import jax.numpy as jnp
import numpy as np
from jax import lax
from jax.experimental import pallas as pl
from jax.experimental.pallas import tpu as pltpu
from jax.experimental.pallas import tpu_sc as plsc

D = 64
NC, NS, L = 2, 16, 16
NW = NC * NS
CHUNK = 128
NFI = 8 * 16

_PERM = np.empty((D,), dtype=np.int32)
for _h in (0, 1):
    for _i in range(16):
        _PERM[_h * 32 + _i] = _h * 32 + 2 * _i
        _PERM[_h * 32 + 16 + _i] = _h * 32 + 2 * _i + 1


def _widen(v32):
    vi = plsc.bitcast(v32, jnp.int32)
    ev = plsc.bitcast(lax.shift_left(vi, 16), jnp.float32)
    od = plsc.bitcast(
        lax.bitwise_and(vi, jnp.int32(-65536)), jnp.float32)
    return ev, od


@functools.cache
def _sc_embed(n_tokens):
    n_per_w = n_tokens // NW
    n_chunks = n_per_w // CHUNK
    mesh = plsc.VectorSubcoreMesh(core_axis_name="c", subcore_axis_name="s")

    @functools.partial(
        pl.kernel,
        out_type=jax.ShapeDtypeStruct((n_tokens, D // 2), jnp.int32),
        mesh=mesh,
        compiler_params=pltpu.CompilerParams(use_tc_tiling_on_sc=False,
                                             needs_layout_passes=False),
        scratch_types=[
            pltpu.VMEM((2, 5, CHUNK), jnp.int32),
            pltpu.VMEM((2, 2, CHUNK, D), jnp.bfloat16),
            pltpu.VMEM((2, CHUNK, D // 2), jnp.int32),
            pltpu.VMEM((2, CHUNK, D // 2), jnp.int32),
            pltpu.VMEM((8, D), jnp.float32),
            pltpu.VMEM((16, D), jnp.float32),
            pltpu.VMEM((NFI, D // 2), jnp.int32),
            pltpu.VMEM_SHARED((4102, D), jnp.bfloat16),
            pltpu.VMEM_SHARED((4102, D), jnp.bfloat16),
            pltpu.VMEM_SHARED((NFI, D // 2), jnp.int32),
            pltpu.SemaphoreType.DMA,
            pltpu.SemaphoreType.DMA,
            pltpu.SemaphoreType.DMA,
            pltpu.SemaphoreType.DMA,
        ],
    )
    def k(comb_hbm, wcx, wcy, wsf, wsi, out_hbm, ib, rows, rows_fi, outb,
          wsf_v, wsi_v, wfi, wcx_sh, wcy_sh, wfi_sh, gsem0, gsem1, osem0,
          osem1):
        wid = lax.axis_index("s") * NC + lax.axis_index("c")
        w_chunk0 = wid * n_chunks
        w_base = wid * n_per_w
        gsems = [gsem0, gsem1]
        osems = [osem0, osem1]
        rnd = jnp.int32(32768)
        himask = jnp.int32(-65536)

        @pl.when(lax.axis_index("s") == 0)
        def _stage():
            pltpu.sync_copy(wcx, wcx_sh)
            pltpu.sync_copy(wcy, wcy_sh)
            pltpu.sync_copy(wsf, wsf_v)
            pltpu.sync_copy(wsi, wsi_v)

            def fuse_body(r, c):
                f = lax.shift_right_logical(r, 4)
                i = lax.bitwise_and(r, 15)
                for h in range(2):
                    ev = (wsf_v[f, pl.ds(h * 32, L)] +
                          wsi_v[i, pl.ds(h * 32, L)])
                    od = (wsf_v[f, pl.ds(h * 32 + L, L)] +
                          wsi_v[i, pl.ds(h * 32 + L, L)])
                    ei = plsc.bitcast(ev, jnp.int32) + rnd
                    oi = plsc.bitcast(od, jnp.int32) + rnd
                    wfi[r, pl.ds(h * L, L)] = lax.bitwise_or(
                        lax.shift_right_logical(ei, 16),
                        lax.bitwise_and(oi, himask))
                return c

            lax.fori_loop(0, NFI, fuse_body, 0)
            pltpu.sync_copy(wfi, wfi_sh)

        plsc.subcore_barrier()

        def load_idx_sync(j, b):
            pltpu.sync_copy(comb_hbm.at[w_chunk0 + j], ib.at[b])

        def mask_idx(b):
            for kk in range(CHUNK // L):
                sl = pl.ds(kk * L, L)
                a = ib[b, 4, sl]
                ib[b, 0, sl] = ib[b, 0, sl] * a
                ib[b, 1, sl] = ib[b, 1, sl] * a
                ib[b, 2, sl] = ib[b, 2, sl] * 16 + ib[b, 3, sl]

        def fire_gathers(b):
            sem = gsems[b]
            pltpu.async_copy(wcx_sh.at[ib.at[b, 0]], rows.at[b, 0], sem)
            pltpu.async_copy(wcy_sh.at[ib.at[b, 1]], rows.at[b, 1], sem)
            pltpu.async_copy(wfi_sh.at[ib.at[b, 2]], rows_fi.at[b], sem)

        def drain_gathers(b):
            sem = gsems[b]
            for t in range(2):
                pltpu.make_async_copy(wcx.at[ib.at[b, t]], rows.at[b, t],
                                      sem).wait()
            pltpu.make_async_copy(wfi_sh.at[ib.at[b, 2]], rows_fi.at[b],
                                  sem).wait()

        def sum_pack(b):
            def body(q, c):
                for rr in range(4):
                    r = q * 4 + rr
                    for h in range(2):
                        vx = rows[b, 0, r, pl.ds(h * 32, 32)]
                        vy = rows[b, 1, r, pl.ds(h * 32, 32)]
                        vf = rows_fi[b, r, pl.ds(h * L, L)]
                        xe, xo = _widen(vx)
                        ye, yo = _widen(vy)
                        fe = plsc.bitcast(lax.shift_left(vf, 16), jnp.float32)
                        fo = plsc.bitcast(lax.bitwise_and(vf, himask),
                                          jnp.float32)
                        se = fe + xe + ye
                        so = fo + xo + yo
                        ei = plsc.bitcast(se, jnp.int32) + rnd
                        oi = plsc.bitcast(so, jnp.int32) + rnd
                        er = lax.shift_right_logical(ei, 16)
                        orr = lax.bitwise_and(oi, himask)
                        outb[b, r, pl.ds(h * L, L)] = lax.bitwise_or(er, orr)
                return c

            lax.fori_loop(0, CHUNK // 4, body, 0)

        def fire_out(j, b):
            base = w_base + j * CHUNK
            pltpu.async_copy(outb.at[b], out_hbm.at[pl.ds(base, CHUNK)],
                             osems[b])

        def drain_out(b):
            pltpu.make_async_copy(outb.at[b],
                                  out_hbm.at[pl.ds(0, CHUNK)],
                                  osems[b]).wait()

        for b in range(2):
            load_idx_sync(b, b)
            mask_idx(b)
            fire_gathers(b)

        def pair_body(t, carry):
            for b in range(2):
                j = t * 2 + b
                drain_gathers(b)
                sum_pack(b)
                fire_out(j, b)
                load_idx_sync(j + 2, b)
                mask_idx(b)
                fire_gathers(b)
                drain_out(b)
            return carry

        lax.fori_loop(0, (n_chunks - 2) // 2, pair_body, 0)

        for b in range(2):
            j = n_chunks - 2 + b
            drain_gathers(b)
            sum_pack(b)
            fire_out(j, b)
        for b in range(2):
            drain_out(b)

    return k


def kernel(cad_vec, flag_vec, index_vec, key_padding_mask, W_cx, W_cy, W_sf,
           W_si):
    B, S = flag_vec.shape
    n = B * S
    nck = n // CHUNK
    x = cad_vec[:, :, 0].reshape(nck, CHUNK)
    y = cad_vec[:, :, 1].reshape(nck, CHUNK)
    fl = flag_vec.reshape(nck, CHUNK)
    iv = index_vec.reshape(nck, CHUNK)
    act = (~key_padding_mask).reshape(nck, CHUNK).astype(jnp.int32)
    comb = jnp.stack([x, y, fl, iv, act], axis=1)
    perm = jnp.asarray(_PERM)
    wcx = W_cx.astype(jnp.bfloat16)
    wcy = W_cy.astype(jnp.bfloat16)
    wsf = W_sf[:, perm]
    wsi = W_si[:, perm]
    out_i32 = _sc_embed(n)(comb, wcx, wcy, wsf, wsi)
    out_bf16 = lax.bitcast_convert_type(out_i32, jnp.bfloat16)
    out = out_bf16.astype(jnp.float32).reshape(n, D)
    return out.reshape(B, S, D)


# speedup vs baseline: 3.6575x; 1.8726x over previous
import functools

import jax
import jax.numpy as jnp
import numpy as np
from jax import lax
from jax.experimental import pallas as pl
from jax.experimental.pallas import tpu as pltpu
from jax.experimental.pallas import tpu_sc as plsc

D = 64
NC, NS, L = 2, 16, 16
NW = NC * NS
CHUNK = 128
NFI = 8 * 16

_PERM = np.empty((D,), dtype=np.int32)
for _h in (0, 1):
    for _i in range(16):
        _PERM[_h * 32 + 2 * _i] = _h * 32 + _i
        _PERM[_h * 32 + 2 * _i + 1] = _h * 32 + 16 + _i


def _widen(v32):
    vi = plsc.bitcast(v32, jnp.int32)
    ev = plsc.bitcast(lax.shift_left(vi, 16), jnp.float32)
    od = plsc.bitcast(
        lax.bitwise_and(vi, jnp.int32(-65536)), jnp.float32)
    return ev, od


@functools.cache
def _sc_embed(n_tokens):
    n_per_w = n_tokens // NW
    n_chunks = n_per_w // CHUNK
    mesh = plsc.VectorSubcoreMesh(core_axis_name="c", subcore_axis_name="s")

    @functools.partial(
        pl.kernel,
        out_type=jax.ShapeDtypeStruct((n_tokens, D), jnp.float32),
        mesh=mesh,
        compiler_params=pltpu.CompilerParams(use_tc_tiling_on_sc=False,
                                             needs_layout_passes=False),
        scratch_types=[
            pltpu.VMEM((2, 5, CHUNK), jnp.int32),
            pltpu.VMEM((2, 2, CHUNK, D), jnp.bfloat16),
            pltpu.VMEM((2, CHUNK, D // 2), jnp.int32),
            pltpu.VMEM((2, CHUNK, D), jnp.float32),
            pltpu.VMEM((8, D), jnp.float32),
            pltpu.VMEM((16, D), jnp.float32),
            pltpu.VMEM((NFI, D // 2), jnp.int32),
            pltpu.VMEM_SHARED((4102, D), jnp.bfloat16),
            pltpu.VMEM_SHARED((4102, D), jnp.bfloat16),
            pltpu.VMEM_SHARED((NFI, D // 2), jnp.int32),
            pltpu.SemaphoreType.DMA,
            pltpu.SemaphoreType.DMA,
            pltpu.SemaphoreType.DMA,
            pltpu.SemaphoreType.DMA,
        ],
    )
    def k(comb_hbm, wcx, wcy, wsf, wsi, out_hbm, ib, rows, rows_fi, outb,
          wsf_v, wsi_v, wfi, wcx_sh, wcy_sh, wfi_sh, gsem0, gsem1, osem0,
          osem1):
        wid = lax.axis_index("s") * NC + lax.axis_index("c")
        w_chunk0 = wid * n_chunks
        w_base = wid * n_per_w
        gsems = [gsem0, gsem1]
        osems = [osem0, osem1]
        rnd = jnp.int32(32768)
        himask = jnp.int32(-65536)

        @pl.when(lax.axis_index("s") == 0)
        def _stage():
            pltpu.sync_copy(wcx, wcx_sh)
            pltpu.sync_copy(wcy, wcy_sh)
            pltpu.sync_copy(wsf, wsf_v)
            pltpu.sync_copy(wsi, wsi_v)

            def fuse_body(r, c):
                f = lax.shift_right_logical(r, 4)
                i = lax.bitwise_and(r, 15)
                for h in range(2):
                    lo = (wsf_v[f, pl.ds(h * 32, L)] +
                          wsi_v[i, pl.ds(h * 32, L)])
                    hi = (wsf_v[f, pl.ds(h * 32 + L, L)] +
                          wsi_v[i, pl.ds(h * 32 + L, L)])
                    li = plsc.bitcast(lo, jnp.int32) + rnd
                    hii = plsc.bitcast(hi, jnp.int32) + rnd
                    wfi[r, pl.ds(h * L, L)] = lax.bitwise_or(
                        lax.shift_right_logical(li, 16),
                        lax.bitwise_and(hii, himask))
                return c

            lax.fori_loop(0, NFI, fuse_body, 0)
            pltpu.sync_copy(wfi, wfi_sh)

        plsc.subcore_barrier()

        def load_idx_sync(j, b):
            pltpu.sync_copy(comb_hbm.at[w_chunk0 + j], ib.at[b])

        def mask_idx(b):
            for kk in range(CHUNK // L):
                sl = pl.ds(kk * L, L)
                a = ib[b, 4, sl]
                ib[b, 0, sl] = ib[b, 0, sl] * a
                ib[b, 1, sl] = ib[b, 1, sl] * a
                ib[b, 2, sl] = ib[b, 2, sl] * 16 + ib[b, 3, sl]

        def fire_gathers(b):
            sem = gsems[b]
            pltpu.async_copy(wcx_sh.at[ib.at[b, 0]], rows.at[b, 0], sem)
            pltpu.async_copy(wcy_sh.at[ib.at[b, 1]], rows.at[b, 1], sem)
            pltpu.async_copy(wfi_sh.at[ib.at[b, 2]], rows_fi.at[b], sem)

        def drain_gathers(b):
            sem = gsems[b]
            for t in range(2):
                pltpu.make_async_copy(wcx.at[ib.at[b, t]], rows.at[b, t],
                                      sem).wait()
            pltpu.make_async_copy(wfi_sh.at[ib.at[b, 2]], rows_fi.at[b],
                                  sem).wait()

        def sum_pack(b):
            def body(q, c):
                for rr in range(4):
                    r = q * 4 + rr
                    for h in range(2):
                        vx = rows[b, 0, r, pl.ds(h * 32, 32)]
                        vy = rows[b, 1, r, pl.ds(h * 32, 32)]
                        vf = rows_fi[b, r, pl.ds(h * L, L)]
                        xe, xo = _widen(vx)
                        ye, yo = _widen(vy)
                        fe = plsc.bitcast(lax.shift_left(vf, 16), jnp.float32)
                        fo = plsc.bitcast(lax.bitwise_and(vf, himask),
                                          jnp.float32)
                        outb[b, r, pl.ds(h * 32, L)] = fe + xe + ye
                        outb[b, r, pl.ds(h * 32 + L, L)] = fo + xo + yo
                return c

            lax.fori_loop(0, CHUNK // 4, body, 0)

        def fire_out(j, b):
            base = w_base + j * CHUNK
            pltpu.async_copy(outb.at[b], out_hbm.at[pl.ds(base, CHUNK)],
                             osems[b])

        def drain_out(b):
            pltpu.make_async_copy(outb.at[b],
                                  out_hbm.at[pl.ds(0, CHUNK)],
                                  osems[b]).wait()

        for b in range(2):
            load_idx_sync(b, b)
            mask_idx(b)
            fire_gathers(b)

        def pair_body(t, carry):
            for b in range(2):
                j = t * 2 + b
                drain_gathers(b)
                sum_pack(b)
                fire_out(j, b)
                load_idx_sync(j + 2, b)
                mask_idx(b)
                fire_gathers(b)
                drain_out(b)
            return carry

        lax.fori_loop(0, (n_chunks - 2) // 2, pair_body, 0)

        for b in range(2):
            j = n_chunks - 2 + b
            drain_gathers(b)
            sum_pack(b)
            fire_out(j, b)
        for b in range(2):
            drain_out(b)

    return k


def kernel(cad_vec, flag_vec, index_vec, key_padding_mask, W_cx, W_cy, W_sf,
           W_si):
    B, S = flag_vec.shape
    n = B * S
    nck = n // CHUNK
    x = cad_vec[:, :, 0].reshape(nck, CHUNK)
    y = cad_vec[:, :, 1].reshape(nck, CHUNK)
    fl = flag_vec.reshape(nck, CHUNK)
    iv = index_vec.reshape(nck, CHUNK)
    act = (~key_padding_mask).reshape(nck, CHUNK).astype(jnp.int32)
    comb = jnp.stack([x, y, fl, iv, act], axis=1)
    perm = jnp.asarray(_PERM)
    wcx = W_cx[:, perm].astype(jnp.bfloat16)
    wcy = W_cy[:, perm].astype(jnp.bfloat16)
    out = _sc_embed(n)(comb, wcx, wcy, W_sf, W_si)
    return out.reshape(B, S, D)


# speedup vs baseline: 3.9617x; 1.0832x over previous
import functools

import jax
import jax.numpy as jnp
import numpy as np
from jax import lax
from jax.experimental import pallas as pl
from jax.experimental.pallas import tpu as pltpu
from jax.experimental.pallas import tpu_sc as plsc

D = 64
NC, NS, L = 2, 16, 16
NW = NC * NS
CHUNK = 128
NFI = 8 * 16

_PERM = np.empty((D,), dtype=np.int32)
for _h in (0, 1):
    for _i in range(16):
        _PERM[_h * 32 + 2 * _i] = _h * 32 + _i
        _PERM[_h * 32 + 2 * _i + 1] = _h * 32 + 16 + _i


def _widen(v32):
    vi = plsc.bitcast(v32, jnp.int32)
    ev = plsc.bitcast(lax.shift_left(vi, 16), jnp.float32)
    od = plsc.bitcast(
        lax.bitwise_and(vi, jnp.int32(-65536)), jnp.float32)
    return ev, od


@functools.cache
def _sc_embed(n_tokens):
    n_per_w = n_tokens // NW
    n_chunks = n_per_w // CHUNK
    mesh = plsc.VectorSubcoreMesh(core_axis_name="c", subcore_axis_name="s")

    @functools.partial(
        pl.kernel,
        out_type=jax.ShapeDtypeStruct((n_tokens, D), jnp.float32),
        mesh=mesh,
        compiler_params=pltpu.CompilerParams(use_tc_tiling_on_sc=False,
                                             needs_layout_passes=False),
        scratch_types=[
            pltpu.VMEM((2, 3, CHUNK), jnp.int32),
            pltpu.VMEM((2, 2, CHUNK, D), jnp.bfloat16),
            pltpu.VMEM((2, CHUNK, D // 2), jnp.int32),
            pltpu.VMEM((2, CHUNK, D), jnp.float32),
            pltpu.VMEM((8, D), jnp.float32),
            pltpu.VMEM((16, D), jnp.float32),
            pltpu.VMEM((NFI, D // 2), jnp.int32),
            pltpu.VMEM_SHARED((4102, D), jnp.bfloat16),
            pltpu.VMEM_SHARED((4102, D), jnp.bfloat16),
            pltpu.VMEM_SHARED((NFI, D // 2), jnp.int32),
            pltpu.SemaphoreType.DMA,
            pltpu.SemaphoreType.DMA,
            pltpu.SemaphoreType.DMA,
            pltpu.SemaphoreType.DMA,
        ],
    )
    def k(comb_hbm, wcx, wcy, wsf, wsi, out_hbm, ib, rows, rows_fi, outb,
          wsf_v, wsi_v, wfi, wcx_sh, wcy_sh, wfi_sh, gsem0, gsem1, osem0,
          osem1):
        wid = lax.axis_index("s") * NC + lax.axis_index("c")
        w_chunk0 = wid * n_chunks
        w_base = wid * n_per_w
        gsems = [gsem0, gsem1]
        osems = [osem0, osem1]
        rnd = jnp.int32(32768)
        himask = jnp.int32(-65536)

        @pl.when(lax.axis_index("s") == 0)
        def _stage():
            pltpu.sync_copy(wcx, wcx_sh)
            pltpu.sync_copy(wcy, wcy_sh)
            pltpu.sync_copy(wsf, wsf_v)
            pltpu.sync_copy(wsi, wsi_v)

            def fuse_body(r, c):
                f = lax.shift_right_logical(r, 4)
                i = lax.bitwise_and(r, 15)
                for h in range(2):
                    lo = (wsf_v[f, pl.ds(h * 32, L)] +
                          wsi_v[i, pl.ds(h * 32, L)])
                    hi = (wsf_v[f, pl.ds(h * 32 + L, L)] +
                          wsi_v[i, pl.ds(h * 32 + L, L)])
                    li = plsc.bitcast(lo, jnp.int32) + rnd
                    hii = plsc.bitcast(hi, jnp.int32) + rnd
                    wfi[r, pl.ds(h * L, L)] = lax.bitwise_or(
                        lax.shift_right_logical(li, 16),
                        lax.bitwise_and(hii, himask))
                return c

            lax.fori_loop(0, NFI, fuse_body, 0)
            pltpu.sync_copy(wfi, wfi_sh)

        plsc.subcore_barrier()

        def load_idx_sync(j, b):
            pltpu.sync_copy(comb_hbm.at[w_chunk0 + j], ib.at[b])

        def fire_gathers(b):
            sem = gsems[b]
            pltpu.async_copy(wcx_sh.at[ib.at[b, 0]], rows.at[b, 0], sem)
            pltpu.async_copy(wcy_sh.at[ib.at[b, 1]], rows.at[b, 1], sem)
            pltpu.async_copy(wfi_sh.at[ib.at[b, 2]], rows_fi.at[b], sem)

        def drain_gathers(b):
            sem = gsems[b]
            for t in range(2):
                pltpu.make_async_copy(wcx.at[ib.at[b, t]], rows.at[b, t],
                                      sem).wait()
            pltpu.make_async_copy(wfi_sh.at[ib.at[b, 2]], rows_fi.at[b],
                                  sem).wait()

        def sum_pack(b):
            def body(q, c):
                for rr in range(4):
                    r = q * 4 + rr
                    for h in range(2):
                        vx = rows[b, 0, r, pl.ds(h * 32, 32)]
                        vy = rows[b, 1, r, pl.ds(h * 32, 32)]
                        vf = rows_fi[b, r, pl.ds(h * L, L)]
                        xe, xo = _widen(vx)
                        ye, yo = _widen(vy)
                        fe = plsc.bitcast(lax.shift_left(vf, 16), jnp.float32)
                        fo = plsc.bitcast(lax.bitwise_and(vf, himask),
                                          jnp.float32)
                        outb[b, r, pl.ds(h * 32, L)] = fe + xe + ye
                        outb[b, r, pl.ds(h * 32 + L, L)] = fo + xo + yo
                return c

            lax.fori_loop(0, CHUNK // 4, body, 0)

        def fire_out(j, b):
            base = w_base + j * CHUNK
            pltpu.async_copy(outb.at[b], out_hbm.at[pl.ds(base, CHUNK)],
                             osems[b])

        def drain_out(b):
            pltpu.make_async_copy(outb.at[b],
                                  out_hbm.at[pl.ds(0, CHUNK)],
                                  osems[b]).wait()

        for b in range(2):
            load_idx_sync(b, b)
            fire_gathers(b)

        def pair_body(t, carry):
            for b in range(2):
                j = t * 2 + b
                drain_gathers(b)
                sum_pack(b)
                fire_out(j, b)
                load_idx_sync(j + 2, b)
                fire_gathers(b)
                drain_out(b)
            return carry

        lax.fori_loop(0, (n_chunks - 2) // 2, pair_body, 0)

        for b in range(2):
            j = n_chunks - 2 + b
            drain_gathers(b)
            sum_pack(b)
            fire_out(j, b)
        for b in range(2):
            drain_out(b)

    return k


def kernel(cad_vec, flag_vec, index_vec, key_padding_mask, W_cx, W_cy, W_sf,
           W_si):
    B, S = flag_vec.shape
    n = B * S
    nck = n // CHUNK
    act = (~key_padding_mask).astype(jnp.int32)
    xm = (cad_vec[:, :, 0] * act).reshape(nck, CHUNK)
    ym = (cad_vec[:, :, 1] * act).reshape(nck, CHUNK)
    fi = (flag_vec * 16 + index_vec).reshape(nck, CHUNK)
    comb = jnp.stack([xm, ym, fi], axis=1)
    perm = jnp.asarray(_PERM)
    wcx = W_cx[:, perm].astype(jnp.bfloat16)
    wcy = W_cy[:, perm].astype(jnp.bfloat16)
    out = _sc_embed(n)(comb, wcx, wcy, W_sf, W_si)
    return out.reshape(B, S, D)


# speedup vs baseline: 4.3917x; 1.1085x over previous
import functools

import jax
import jax.numpy as jnp
import numpy as np
from jax import lax
from jax.experimental import pallas as pl
from jax.experimental.pallas import tpu as pltpu
from jax.experimental.pallas import tpu_sc as plsc

D = 64
NC, NS, L = 2, 16, 16
NW = NC * NS
CHUNK = 128
NFI = 8 * 16

_PERM = np.empty((D,), dtype=np.int32)
for _h in (0, 1):
    for _i in range(16):
        _PERM[_h * 32 + 2 * _i] = _h * 32 + _i
        _PERM[_h * 32 + 2 * _i + 1] = _h * 32 + 16 + _i


def _widen(v32):
    vi = plsc.bitcast(v32, jnp.int32)
    ev = plsc.bitcast(lax.shift_left(vi, 16), jnp.float32)
    od = plsc.bitcast(
        lax.bitwise_and(vi, jnp.int32(-65536)), jnp.float32)
    return ev, od


@functools.cache
def _sc_embed(n_tokens):
    n_per_w = n_tokens // NW
    n_chunks = n_per_w // CHUNK
    mesh = plsc.VectorSubcoreMesh(core_axis_name="c", subcore_axis_name="s")

    @functools.partial(
        pl.kernel,
        out_type=jax.ShapeDtypeStruct((n_tokens, D), jnp.float32),
        mesh=mesh,
        compiler_params=pltpu.CompilerParams(use_tc_tiling_on_sc=False,
                                             needs_layout_passes=False),
        scratch_types=[
            pltpu.VMEM((4, 3, CHUNK), jnp.int32),
            pltpu.VMEM((2, 2, CHUNK, D), jnp.bfloat16),
            pltpu.VMEM((2, CHUNK, D // 2), jnp.int32),
            pltpu.VMEM((2, CHUNK, D), jnp.float32),
            pltpu.VMEM((8, D), jnp.float32),
            pltpu.VMEM((16, D), jnp.float32),
            pltpu.VMEM((NFI, D // 2), jnp.int32),
            pltpu.VMEM_SHARED((4102, D), jnp.bfloat16),
            pltpu.VMEM_SHARED((4102, D), jnp.bfloat16),
            pltpu.VMEM_SHARED((NFI, D // 2), jnp.int32),
            pltpu.SemaphoreType.DMA,
            pltpu.SemaphoreType.DMA,
            pltpu.SemaphoreType.DMA,
            pltpu.SemaphoreType.DMA,
            pltpu.SemaphoreType.DMA,
            pltpu.SemaphoreType.DMA,
            pltpu.SemaphoreType.DMA,
            pltpu.SemaphoreType.DMA,
        ],
    )
    def k(comb_hbm, wcx, wcy, wsf, wsi, out_hbm, ib, rows, rows_fi, outb,
          wsf_v, wsi_v, wfi, wcx_sh, wcy_sh, wfi_sh, gsem0, gsem1, osem0,
          osem1, isem0, isem1, isem2, isem3):
        wid = lax.axis_index("s") * NC + lax.axis_index("c")
        w_chunk0 = wid * n_chunks
        w_base = wid * n_per_w
        gsems = [gsem0, gsem1]
        osems = [osem0, osem1]
        isems = [isem0, isem1, isem2, isem3]
        rnd = jnp.int32(32768)
        himask = jnp.int32(-65536)

        @pl.when(lax.axis_index("s") == 0)
        def _stage():
            pltpu.sync_copy(wcx, wcx_sh)
            pltpu.sync_copy(wcy, wcy_sh)
            pltpu.sync_copy(wsf, wsf_v)
            pltpu.sync_copy(wsi, wsi_v)

            def fuse_body(r, c):
                f = lax.shift_right_logical(r, 4)
                i = lax.bitwise_and(r, 15)
                for h in range(2):
                    lo = (wsf_v[f, pl.ds(h * 32, L)] +
                          wsi_v[i, pl.ds(h * 32, L)])
                    hi = (wsf_v[f, pl.ds(h * 32 + L, L)] +
                          wsi_v[i, pl.ds(h * 32 + L, L)])
                    li = plsc.bitcast(lo, jnp.int32) + rnd
                    hii = plsc.bitcast(hi, jnp.int32) + rnd
                    wfi[r, pl.ds(h * L, L)] = lax.bitwise_or(
                        lax.shift_right_logical(li, 16),
                        lax.bitwise_and(hii, himask))
                return c

            lax.fori_loop(0, NFI, fuse_body, 0)
            pltpu.sync_copy(wfi, wfi_sh)

        plsc.subcore_barrier()

        def load_idx_sync(j, s):
            pltpu.sync_copy(comb_hbm.at[w_chunk0 + j], ib.at[s])

        def fire_idx(j, s):
            pltpu.async_copy(comb_hbm.at[w_chunk0 + j], ib.at[s], isems[s])

        def wait_idx(s):
            pltpu.make_async_copy(comb_hbm.at[w_chunk0], ib.at[s],
                                  isems[s]).wait()

        def fire_gathers(s, p):
            sem = gsems[p]
            pltpu.async_copy(wcx_sh.at[ib.at[s, 0]], rows.at[p, 0], sem)
            pltpu.async_copy(wcy_sh.at[ib.at[s, 1]], rows.at[p, 1], sem)
            pltpu.async_copy(wfi_sh.at[ib.at[s, 2]], rows_fi.at[p], sem)

        def drain_gathers(s, p):
            sem = gsems[p]
            for t in range(2):
                pltpu.make_async_copy(wcx.at[ib.at[s, t]], rows.at[p, t],
                                      sem).wait()
            pltpu.make_async_copy(wfi_sh.at[ib.at[s, 2]], rows_fi.at[p],
                                  sem).wait()

        def sum_pack(p):
            def body(q, c):
                for rr in range(4):
                    r = q * 4 + rr
                    for h in range(2):
                        vx = rows[p, 0, r, pl.ds(h * 32, 32)]
                        vy = rows[p, 1, r, pl.ds(h * 32, 32)]
                        vf = rows_fi[p, r, pl.ds(h * L, L)]
                        xe, xo = _widen(vx)
                        ye, yo = _widen(vy)
                        fe = plsc.bitcast(lax.shift_left(vf, 16), jnp.float32)
                        fo = plsc.bitcast(lax.bitwise_and(vf, himask),
                                          jnp.float32)
                        outb[p, r, pl.ds(h * 32, L)] = fe + xe + ye
                        outb[p, r, pl.ds(h * 32 + L, L)] = fo + xo + yo
                return c

            lax.fori_loop(0, CHUNK // 4, body, 0)

        def fire_out(j, p):
            base = w_base + j * CHUNK
            pltpu.async_copy(outb.at[p], out_hbm.at[pl.ds(base, CHUNK)],
                             osems[p])

        def drain_out(p):
            pltpu.make_async_copy(outb.at[p],
                                  out_hbm.at[pl.ds(0, CHUNK)],
                                  osems[p]).wait()

        for c in range(2):
            load_idx_sync(c, c)
            fire_gathers(c, c)
        for c in range(2, 4):
            fire_idx(c, c)
        for c in range(2):
            drain_gathers(c, c)
            sum_pack(c)
            fire_out(c, c)
            wait_idx(c + 2)
            fire_gathers(c + 2, c)
            fire_idx(c + 4, c)

        def quad_body(t, carry):
            for b in range(4):
                j = 2 + t * 4 + b
                s_cur = (2 + b) % 4
                s_nxt = b
                p = b % 2
                drain_gathers(s_cur, p)
                drain_out(p)
                sum_pack(p)
                fire_out(j, p)
                wait_idx(s_nxt)
                fire_gathers(s_nxt, p)

                @pl.when(j + 4 < n_chunks)
                def _pf():
                    fire_idx(j + 4, s_cur)
            return carry

        lax.fori_loop(0, (n_chunks - 4) // 4, quad_body, 0)

        for b in range(2):
            j = n_chunks - 2 + b
            drain_gathers(j % 4, b)
            drain_out(b)
            sum_pack(b)
            fire_out(j, b)
        for b in range(2):
            drain_out(b)

    return k


def kernel(cad_vec, flag_vec, index_vec, key_padding_mask, W_cx, W_cy, W_sf,
           W_si):
    B, S = flag_vec.shape
    n = B * S
    nck = n // CHUNK
    act = (~key_padding_mask).astype(jnp.int32)
    xm = (cad_vec[:, :, 0] * act).reshape(nck, CHUNK)
    ym = (cad_vec[:, :, 1] * act).reshape(nck, CHUNK)
    fi = (flag_vec * 16 + index_vec).reshape(nck, CHUNK)
    comb = jnp.stack([xm, ym, fi], axis=1)
    perm = jnp.asarray(_PERM)
    wcx = W_cx[:, perm].astype(jnp.bfloat16)
    wcy = W_cy[:, perm].astype(jnp.bfloat16)
    out = _sc_embed(n)(comb, wcx, wcy, W_sf, W_si)
    return out.reshape(B, S, D)
